# baseline (speedup 1.0000x reference)
import functools
import math

import jax
import jax.numpy as jnp
from jax import lax
from jax.experimental import pallas as pl
from jax.experimental.pallas import tpu as pltpu
from jax.experimental.pallas import tpu_sc as plsc

D_MODEL = 64
SCALE = math.sqrt(D_MODEL)

NC = 2
NS = 16
NW = NC * NS
LANES = 16

CHUNK = 128
NBUF = 4
TR = 8


def _sc_embed(idx_t, lut, *, n_seq, n_batch):
    n_chunks = n_seq
    n_blocks = n_batch // CHUNK
    d_tiles = D_MODEL // TR

    mesh = plsc.VectorSubcoreMesh(core_axis_name="c", subcore_axis_name="s")

    @functools.partial(
        pl.kernel,
        mesh=mesh,
        out_type=jax.ShapeDtypeStruct(
            (n_seq, d_tiles, n_blocks, TR, CHUNK), jnp.float32
        ),
        scratch_types=[
            pltpu.VMEM((n_chunks, CHUNK), jnp.int32),
            pltpu.VMEM((NBUF, CHUNK, D_MODEL), jnp.float32),
            pltpu.VMEM((NBUF, d_tiles, TR, CHUNK), jnp.float32),
            pltpu.SemaphoreType.DMA((NBUF,)),
            pltpu.SemaphoreType.DMA((NBUF,)),
        ],
        compiler_params=pltpu.CompilerParams(
            use_tc_tiling_on_sc=False, needs_layout_passes=False
        ),
    )
    def k(lut_hbm, idx_hbm, out_hbm, idx_v, raw_v, out_v, gsem, ssem):
        wid = lax.axis_index("s") * NC + lax.axis_index("c")
        pltpu.sync_copy(idx_hbm.at[:, pl.ds(wid * CHUNK, CHUNK)], idx_v)

        def gather_start(c, b):
            pltpu.make_async_copy(
                lut_hbm.at[idx_v.at[c]], raw_v.at[b], gsem.at[b]
            ).start()

        def gather_wait(c, b):
            pltpu.make_async_copy(
                lut_hbm.at[idx_v.at[c]], raw_v.at[b], gsem.at[b]
            ).wait()

        def store_start(c, b):
            pltpu.make_async_copy(
                out_v.at[b], out_hbm.at[c, :, wid], ssem.at[b]
            ).start()

        def store_wait(b):
            pltpu.make_async_copy(
                out_v.at[b], out_hbm.at[0, :, 0], ssem.at[b]
            ).wait()

        lane = lax.iota(jnp.int32, LANES)

        for b in range(NBUF - 1):
            gather_start(b, b)

        def group_body(g, _):
            for b in range(NBUF):
                c = g * NBUF + b

                @pl.when(c + NBUF - 1 < n_chunks)
                def _(c=c, b=b):
                    gather_start(c + NBUF - 1, (b + NBUF - 1) % NBUF)

                gather_wait(c, b)

                @pl.when(c >= NBUF)
                def _(b=b):
                    store_wait(b)

                @plsc.parallel_loop(0, D_MODEL, step=1, unroll=2)
                def _(d, b=b):
                    col = jnp.full((LANES,), d, jnp.int32)
                    for grp in range(CHUNK // LANES):
                        rows = lane + (grp * LANES)
                        vals = plsc.load_gather(raw_v.at[b], [rows, col])
                        out_v[b, d // TR, d % TR, pl.ds(grp * LANES, LANES)] = (
                            vals * SCALE
                        )

                store_start(c, b)
            return ()

        lax.fori_loop(0, n_chunks // NBUF, group_body, ())

        for b in range(NBUF):
            store_wait(b)

    return k(lut, idx_t)


def kernel(x, lut):
    n_batch, n_seq = x.shape
    idx_t = jnp.swapaxes(x, 0, 1).astype(jnp.int32)
    out5d = _sc_embed(idx_t, lut, n_seq=n_seq, n_batch=n_batch)
    out = out5d.transpose(2, 4, 0, 1, 3).reshape(n_batch, n_seq, D_MODEL)
    return out


# speedup vs baseline: 1.6768x; 1.6768x over previous
import functools
import math

import jax
import jax.numpy as jnp
from jax import lax
from jax.experimental import pallas as pl
from jax.experimental.pallas import tpu as pltpu
from jax.experimental.pallas import tpu_sc as plsc

D_MODEL = 64
SCALE = math.sqrt(D_MODEL)

NC = 2
NS = 16
NW = NC * NS
LANES = 16

CHUNK = 128
NBUF = 4
TR = 8


def _sc_embed(idx_t, lut, *, n_seq, n_batch):
    n_chunks = n_seq
    n_blocks = n_batch // CHUNK
    d_tiles = D_MODEL // TR

    mesh = plsc.VectorSubcoreMesh(core_axis_name="c", subcore_axis_name="s")

    @functools.partial(
        pl.kernel,
        mesh=mesh,
        out_type=jax.ShapeDtypeStruct(
            (n_seq, d_tiles, n_blocks, TR, CHUNK), jnp.float32
        ),
        scratch_types=[
            pltpu.VMEM((n_chunks, CHUNK), jnp.int32),
            pltpu.VMEM((NBUF, CHUNK, D_MODEL), jnp.float32),
            pltpu.VMEM((NBUF, d_tiles, TR, CHUNK + 1), jnp.float32),
            pltpu.SemaphoreType.DMA((NBUF,)),
            pltpu.SemaphoreType.DMA((NBUF,)),
        ],
        compiler_params=pltpu.CompilerParams(
            use_tc_tiling_on_sc=False, needs_layout_passes=False
        ),
    )
    def k(lut_hbm, idx_hbm, out_hbm, idx_v, raw_v, out_v, gsem, ssem):
        wid = lax.axis_index("s") * NC + lax.axis_index("c")
        pltpu.sync_copy(idx_hbm.at[:, pl.ds(wid * CHUNK, CHUNK)], idx_v)

        def gather_start(c, b):
            pltpu.make_async_copy(
                lut_hbm.at[idx_v.at[c]], raw_v.at[b], gsem.at[b]
            ).start()

        def gather_wait(c, b):
            pltpu.make_async_copy(
                lut_hbm.at[idx_v.at[c]], raw_v.at[b], gsem.at[b]
            ).wait()

        def store_start(c, b):
            pltpu.make_async_copy(
                out_v.at[b, :, :, pl.ds(0, CHUNK)],
                out_hbm.at[c, :, wid], ssem.at[b],
            ).start()

        def store_wait(b):
            pltpu.make_async_copy(
                out_v.at[b, :, :, pl.ds(0, CHUNK)],
                out_hbm.at[0, :, 0], ssem.at[b],
            ).wait()

        lane = lax.iota(jnp.int32, LANES)

        for b in range(NBUF - 1):
            gather_start(b, b)

        def group_body(g, _):
            for b in range(NBUF):
                c = g * NBUF + b

                @pl.when(c + NBUF - 1 < n_chunks)
                def _(c=c, b=b):
                    gather_start(c + NBUF - 1, (b + NBUF - 1) % NBUF)

                gather_wait(c, b)

                @pl.when(c >= NBUF)
                def _(b=b):
                    store_wait(b)

                @plsc.parallel_loop(0, CHUNK, step=1, unroll=4)
                def _(i, b=b):
                    col = jnp.full((LANES,), i, jnp.int32)
                    for grp in range(D_MODEL // LANES):
                        d = lane + (grp * LANES)
                        vals = raw_v[b, i, pl.ds(grp * LANES, LANES)] * SCALE
                        plsc.store_scatter(
                            out_v.at[b],
                            [d // TR, d % TR, col],
                            vals,
                        )

                store_start(c, b)
            return ()

        lax.fori_loop(0, n_chunks // NBUF, group_body, ())

        for b in range(NBUF):
            store_wait(b)

    return k(lut, idx_t)


def kernel(x, lut):
    n_batch, n_seq = x.shape
    idx_t = jnp.swapaxes(x, 0, 1).astype(jnp.int32)
    out5d = _sc_embed(idx_t, lut, n_seq=n_seq, n_batch=n_batch)
    out = out5d.transpose(2, 4, 0, 1, 3).reshape(n_batch, n_seq, D_MODEL)
    return out


# speedup vs baseline: 1.6784x; 1.0009x over previous
import functools
import math

import jax
import jax.numpy as jnp
from jax import lax
from jax.experimental import pallas as pl
from jax.experimental.pallas import tpu as pltpu
from jax.experimental.pallas import tpu_sc as plsc

D_MODEL = 64
SCALE = math.sqrt(D_MODEL)

NC = 2
NS = 16
NW = NC * NS
LANES = 16

CHUNK = 128
NBUF = 4
TR = 8
IPITCH = CHUNK + 1


def _sc_embed(x_flat, lut, *, n_seq, n_batch):
    n_chunks = n_seq
    n_blocks = n_batch // CHUNK
    n_per_w = n_seq * CHUNK
    d_tiles = D_MODEL // TR

    mesh = plsc.VectorSubcoreMesh(core_axis_name="c", subcore_axis_name="s")

    @functools.partial(
        pl.kernel,
        mesh=mesh,
        out_type=jax.ShapeDtypeStruct(
            (n_seq, d_tiles, n_blocks, TR, CHUNK), jnp.float32
        ),
        scratch_types=[
            pltpu.VMEM((n_per_w,), jnp.int32),
            pltpu.VMEM((n_chunks, IPITCH), jnp.int32),
            pltpu.VMEM((NBUF, CHUNK, D_MODEL), jnp.float32),
            pltpu.VMEM((NBUF, d_tiles, TR, IPITCH), jnp.float32),
            pltpu.SemaphoreType.DMA((NBUF,)),
            pltpu.SemaphoreType.DMA((NBUF,)),
        ],
        compiler_params=pltpu.CompilerParams(
            use_tc_tiling_on_sc=False, needs_layout_passes=False
        ),
    )
    def k(lut_hbm, idx_hbm, out_hbm, idx_v, idx2_v, raw_v, out_v, gsem, ssem):
        wid = lax.axis_index("s") * NC + lax.axis_index("c")
        pltpu.sync_copy(idx_hbm.at[pl.ds(wid * n_per_w, n_per_w)], idx_v)

        lane = lax.iota(jnp.int32, LANES)

        def repack(c):
            for q in range(CHUNK // LANES):
                pick = (lane + q * LANES) * n_seq + c
                idx2_v[c, pl.ds(q * LANES, LANES)] = plsc.load_gather(
                    idx_v, [pick]
                )

        def gather_start(c, b):
            pltpu.make_async_copy(
                lut_hbm.at[idx2_v.at[c, pl.ds(0, CHUNK)]],
                raw_v.at[b], gsem.at[b],
            ).start()

        def gather_wait(c, b):
            pltpu.make_async_copy(
                lut_hbm.at[idx2_v.at[c, pl.ds(0, CHUNK)]],
                raw_v.at[b], gsem.at[b],
            ).wait()

        def store_start(c, b):
            pltpu.make_async_copy(
                out_v.at[b, :, :, pl.ds(0, CHUNK)],
                out_hbm.at[c, :, wid], ssem.at[b],
            ).start()

        def store_wait(b):
            pltpu.make_async_copy(
                out_v.at[b, :, :, pl.ds(0, CHUNK)],
                out_hbm.at[0, :, 0], ssem.at[b],
            ).wait()

        for b in range(NBUF - 1):
            repack(b)
            gather_start(b, b)

        def group_body(g, _):
            for b in range(NBUF):
                c = g * NBUF + b

                @pl.when(c + NBUF - 1 < n_chunks)
                def _(c=c, b=b):
                    repack(c + NBUF - 1)
                    gather_start(c + NBUF - 1, (b + NBUF - 1) % NBUF)

                gather_wait(c, b)

                @pl.when(c >= NBUF)
                def _(b=b):
                    store_wait(b)

                @plsc.parallel_loop(0, CHUNK, step=1, unroll=4)
                def _(i, b=b):
                    col = jnp.full((LANES,), i, jnp.int32)
                    for grp in range(D_MODEL // LANES):
                        d = lane + (grp * LANES)
                        vals = raw_v[b, i, pl.ds(grp * LANES, LANES)] * SCALE
                        plsc.store_scatter(
                            out_v.at[b],
                            [d // TR, d % TR, col],
                            vals,
                        )

                store_start(c, b)
            return ()

        lax.fori_loop(0, n_chunks // NBUF, group_body, ())

        for b in range(NBUF):
            store_wait(b)

    return k(lut, x_flat)


def kernel(x, lut):
    n_batch, n_seq = x.shape
    x_flat = x.reshape(n_batch * n_seq).astype(jnp.int32)
    out5d = _sc_embed(x_flat, lut, n_seq=n_seq, n_batch=n_batch)
    out = out5d.transpose(2, 4, 0, 1, 3).reshape(n_batch, n_seq, D_MODEL)
    return out


# speedup vs baseline: 3.0109x; 1.7939x over previous
import functools
import math

import jax
import jax.numpy as jnp
from jax import lax
from jax.experimental import pallas as pl
from jax.experimental.pallas import tpu as pltpu
from jax.experimental.pallas import tpu_sc as plsc

D_MODEL = 64
SCALE = math.sqrt(D_MODEL)

NC = 2
NS = 16
NW = NC * NS
LANES = 16

CHUNK = 128
NBUF = 4
TR = 8
IPITCH = CHUNK + 1

VBLK = 128
NB1 = 4
DMASK = D_MODEL - 1
RMASK = LANES - 1


def _sc_detile(lut_t, tail_pairs, *, vocab):
    n_tblk = vocab // VBLK
    tail_w = vocab - n_tblk * VBLK
    iters_max = (n_tblk + NW - 1) // NW
    groups = (iters_max + NB1 - 1) // NB1

    mesh = plsc.VectorSubcoreMesh(core_axis_name="c", subcore_axis_name="s")

    @functools.partial(
        pl.kernel,
        mesh=mesh,
        out_type=jax.ShapeDtypeStruct((vocab // 2, 2 * D_MODEL), jnp.float32),
        scratch_types=[
            pltpu.VMEM((NB1, D_MODEL, VBLK), jnp.float32),
            pltpu.VMEM((NB1, VBLK // 2, 2 * D_MODEL), jnp.float32),
            pltpu.SemaphoreType.DMA((NB1,)),
            pltpu.SemaphoreType.DMA((NB1,)),
        ],
        compiler_params=pltpu.CompilerParams(
            use_tc_tiling_on_sc=True, needs_layout_passes=False
        ),
    )
    def k1(lut_hbm, tail_hbm, out_hbm, in_v, st_v, gsem, ssem):
        wid = lax.axis_index("s") * NC + lax.axis_index("c")
        iters_w = (n_tblk - wid + NW - 1) // NW

        lane = lax.iota(jnp.int32, LANES)

        def blk_of(i):
            return i * NW + wid

        def load_start(blk, b):
            pltpu.make_async_copy(
                lut_hbm.at[:, pl.ds(blk * VBLK, VBLK)], in_v.at[b], gsem.at[b]
            ).start()

        def load_wait(blk, b):
            pltpu.make_async_copy(
                lut_hbm.at[:, pl.ds(0, VBLK)], in_v.at[b], gsem.at[b]
            ).wait()

        def store_start(blk, b):
            pltpu.make_async_copy(
                st_v.at[b],
                out_hbm.at[pl.ds(blk * (VBLK // 2), VBLK // 2)], ssem.at[b],
            ).start()

        def store_wait(blk, b):
            pltpu.make_async_copy(
                st_v.at[b], out_hbm.at[pl.ds(0, VBLK // 2)], ssem.at[b]
            ).wait()

        @pl.when(wid == 0)
        def _():
            pltpu.sync_copy(tail_hbm, st_v.at[0, pl.ds(0, tail_w // 2)])
            pltpu.sync_copy(
                st_v.at[0, pl.ds(0, tail_w // 2)],
                out_hbm.at[pl.ds(vocab // 2 - tail_w // 2, tail_w // 2)],
            )

        for b in range(NB1 - 1):
            @pl.when(b < iters_w)
            def _(b=b):
                load_start(blk_of(b), b)

        def body(g, _):
            for b in range(NB1):
                i = g * NB1 + b

                @pl.when(i + NB1 - 1 < iters_w)
                def _(i=i, b=b):
                    load_start(blk_of(i + NB1 - 1), (b + NB1 - 1) % NB1)

                @pl.when(i < iters_w)
                def _(i=i, b=b):
                    blk = blk_of(i)
                    load_wait(blk, b)

                    @pl.when(i >= NB1)
                    def _():
                        store_wait(blk_of(i - NB1), b)

                    @plsc.parallel_loop(0, D_MODEL, step=1, unroll=2)
                    def _(d):
                        for q in range(VBLK // LANES):
                            v = lane + q * LANES
                            col = (v & 1) * D_MODEL + ((d + (v & RMASK)) & DMASK)
                            vals = in_v[b, d, pl.ds(q * LANES, LANES)]
                            plsc.store_scatter(
                                st_v.at[b], [v // 2, col], vals
                            )

                    store_start(blk, b)
            return ()

        lax.fori_loop(0, groups, body, ())

        for b in range(NB1):
            i_b = iters_w - 1 - lax.rem(iters_w - 1 - b + 4 * NB1, NB1)

            @pl.when(i_b >= 0)
            def _(i_b=i_b, b=b):
                store_wait(blk_of(i_b), b)

    return k1(lut_t, tail_pairs)


def _sc_embed(x_flat, tbl, *, n_seq, n_batch, vocab):
    n_chunks = n_seq
    n_blocks = n_batch // CHUNK
    n_per_w = n_seq * CHUNK
    d_tiles = D_MODEL // TR

    mesh = plsc.VectorSubcoreMesh(core_axis_name="c", subcore_axis_name="s")

    @functools.partial(
        pl.kernel,
        mesh=mesh,
        out_type=jax.ShapeDtypeStruct(
            (n_seq, d_tiles, n_blocks, TR, CHUNK), jnp.float32
        ),
        scratch_types=[
            pltpu.VMEM((n_per_w,), jnp.int32),
            pltpu.VMEM((n_chunks, IPITCH), jnp.int32),
            pltpu.VMEM((NBUF, CHUNK, D_MODEL), jnp.float32),
            pltpu.VMEM((NBUF, d_tiles, TR, IPITCH), jnp.float32),
            pltpu.SemaphoreType.DMA((NBUF,)),
            pltpu.SemaphoreType.DMA((NBUF,)),
        ],
        compiler_params=pltpu.CompilerParams(
            use_tc_tiling_on_sc=False, needs_layout_passes=False
        ),
    )
    def k(tbl_hbm, idx_hbm, out_hbm, idx_v, idx2_v, raw_v, out_v, gsem, ssem):
        wid = lax.axis_index("s") * NC + lax.axis_index("c")
        pltpu.sync_copy(idx_hbm.at[pl.ds(wid * n_per_w, n_per_w)], idx_v)

        lane = lax.iota(jnp.int32, LANES)

        def repack(c):
            for q in range(CHUNK // LANES):
                pick = (lane + q * LANES) * n_seq + c
                idx2_v[c, pl.ds(q * LANES, LANES)] = plsc.load_gather(
                    idx_v, [pick]
                )

        def gather_start(c, b):
            pltpu.make_async_copy(
                tbl_hbm.at[idx2_v.at[c, pl.ds(0, CHUNK)]],
                raw_v.at[b], gsem.at[b],
            ).start()

        def gather_wait(c, b):
            pltpu.make_async_copy(
                tbl_hbm.at[idx2_v.at[c, pl.ds(0, CHUNK)]],
                raw_v.at[b], gsem.at[b],
            ).wait()

        def store_start(c, b):
            pltpu.make_async_copy(
                out_v.at[b, :, :, pl.ds(0, CHUNK)],
                out_hbm.at[c, :, wid], ssem.at[b],
            ).start()

        def store_wait(b):
            pltpu.make_async_copy(
                out_v.at[b, :, :, pl.ds(0, CHUNK)],
                out_hbm.at[0, :, 0], ssem.at[b],
            ).wait()

        for b in range(NBUF - 1):
            repack(b)
            gather_start(b, b)

        def group_body(g, _):
            for b in range(NBUF):
                c = g * NBUF + b

                @pl.when(c + NBUF - 1 < n_chunks)
                def _(c=c, b=b):
                    repack(c + NBUF - 1)
                    gather_start(c + NBUF - 1, (b + NBUF - 1) % NBUF)

                gather_wait(c, b)

                @pl.when(c >= NBUF)
                def _(b=b):
                    store_wait(b)

                @plsc.parallel_loop(0, CHUNK, step=1, unroll=4)
                def _(i, c=c, b=b):
                    col = jnp.full((LANES,), i, jnp.int32)
                    rot = plsc.load_gather(idx2_v.at[c], [col]) & RMASK
                    for grp in range(D_MODEL // LANES):
                        q = lane + (grp * LANES)
                        d = (q - rot) & DMASK
                        vals = raw_v[b, i, pl.ds(grp * LANES, LANES)] * SCALE
                        plsc.store_scatter(
                            out_v.at[b],
                            [d // TR, d % TR, col],
                            vals,
                        )

                store_start(c, b)
            return ()

        lax.fori_loop(0, n_chunks // NBUF, group_body, ())

        for b in range(NBUF):
            store_wait(b)

    return k(tbl, x_flat)


def kernel(x, lut):
    n_batch, n_seq = x.shape
    vocab = lut.shape[0]
    x_flat = x.reshape(n_batch * n_seq).astype(jnp.int32)
    n_tail = vocab - (vocab // VBLK) * VBLK
    t = lut[vocab - n_tail:].reshape(n_tail // LANES, LANES, D_MODEL)
    tail_skewed = jnp.stack(
        [jnp.roll(t[:, r], r, axis=-1) for r in range(LANES)], axis=1
    ).reshape(n_tail, D_MODEL)
    tail_pairs = tail_skewed.reshape(n_tail // 2, 2 * D_MODEL)
    tbl_pairs = _sc_detile(jnp.swapaxes(lut, 0, 1), tail_pairs, vocab=vocab)
    tbl = tbl_pairs.reshape(vocab * D_MODEL).reshape(vocab, D_MODEL)
    out5d = _sc_embed(
        x_flat, tbl, n_seq=n_seq, n_batch=n_batch, vocab=vocab
    )
    out = out5d.transpose(2, 4, 0, 1, 3).reshape(n_batch, n_seq, D_MODEL)
    return out


# speedup vs baseline: 3.1511x; 1.0466x over previous
import functools
import math

import jax
import jax.numpy as jnp
from jax import lax
from jax.experimental import pallas as pl
from jax.experimental.pallas import tpu as pltpu
from jax.experimental.pallas import tpu_sc as plsc

D_MODEL = 64
SCALE = math.sqrt(D_MODEL)

NC = 2
NS = 16
NW = NC * NS
LANES = 16

CHUNK = 128
NBUF = 4
TR = 8
IPITCH = CHUNK + 1

VBLK = 256
NB1 = 3
DMASK = D_MODEL - 1
RMASK = LANES - 1


def _sc_detile(lut_t, tail_pairs, *, vocab):
    n_tblk = vocab // VBLK
    tail_w = vocab - n_tblk * VBLK
    iters_max = (n_tblk + NW - 1) // NW
    groups = (iters_max + NB1 - 1) // NB1

    mesh = plsc.VectorSubcoreMesh(core_axis_name="c", subcore_axis_name="s")

    @functools.partial(
        pl.kernel,
        mesh=mesh,
        out_type=jax.ShapeDtypeStruct((vocab // 2, 2 * D_MODEL), jnp.float32),
        scratch_types=[
            pltpu.VMEM((NB1, D_MODEL, VBLK), jnp.float32),
            pltpu.VMEM((NB1, VBLK // 2, 2 * D_MODEL), jnp.float32),
            pltpu.SemaphoreType.DMA((NB1,)),
            pltpu.SemaphoreType.DMA((NB1,)),
        ],
        compiler_params=pltpu.CompilerParams(
            use_tc_tiling_on_sc=True, needs_layout_passes=False
        ),
    )
    def k1(lut_hbm, tail_hbm, out_hbm, in_v, st_v, gsem, ssem):
        wid = lax.axis_index("s") * NC + lax.axis_index("c")
        iters_w = (n_tblk - wid + NW - 1) // NW

        lane = lax.iota(jnp.int32, LANES)

        def blk_of(i):
            return i * NW + wid

        def load_start(blk, b):
            pltpu.make_async_copy(
                lut_hbm.at[:, pl.ds(blk * VBLK, VBLK)], in_v.at[b], gsem.at[b]
            ).start()

        def load_wait(blk, b):
            pltpu.make_async_copy(
                lut_hbm.at[:, pl.ds(0, VBLK)], in_v.at[b], gsem.at[b]
            ).wait()

        def store_start(blk, b):
            pltpu.make_async_copy(
                st_v.at[b],
                out_hbm.at[pl.ds(blk * (VBLK // 2), VBLK // 2)], ssem.at[b],
            ).start()

        def store_wait(blk, b):
            pltpu.make_async_copy(
                st_v.at[b], out_hbm.at[pl.ds(0, VBLK // 2)], ssem.at[b]
            ).wait()

        @pl.when(wid == 0)
        def _():
            pltpu.sync_copy(tail_hbm, st_v.at[0, pl.ds(0, tail_w // 2)])
            pltpu.sync_copy(
                st_v.at[0, pl.ds(0, tail_w // 2)],
                out_hbm.at[pl.ds(vocab // 2 - tail_w // 2, tail_w // 2)],
            )

        for b in range(NB1 - 1):
            @pl.when(b < iters_w)
            def _(b=b):
                load_start(blk_of(b), b)

        def body(g, _):
            for b in range(NB1):
                i = g * NB1 + b

                @pl.when(i + NB1 - 1 < iters_w)
                def _(i=i, b=b):
                    load_start(blk_of(i + NB1 - 1), (b + NB1 - 1) % NB1)

                @pl.when(i < iters_w)
                def _(i=i, b=b):
                    blk = blk_of(i)
                    load_wait(blk, b)

                    @pl.when(i >= NB1)
                    def _():
                        store_wait(blk_of(i - NB1), b)

                    @plsc.parallel_loop(0, D_MODEL, step=1, unroll=2)
                    def _(d):
                        for q in range(VBLK // LANES):
                            v = lane + q * LANES
                            col = (v & 1) * D_MODEL + ((d + (v & RMASK)) & DMASK)
                            vals = in_v[b, d, pl.ds(q * LANES, LANES)] * SCALE
                            plsc.store_scatter(
                                st_v.at[b], [v // 2, col], vals
                            )

                    store_start(blk, b)
            return ()

        lax.fori_loop(0, groups, body, ())

        for b in range(NB1):
            i_b = iters_w - 1 - lax.rem(iters_w - 1 - b + 4 * NB1, NB1)

            @pl.when(i_b >= 0)
            def _(i_b=i_b, b=b):
                store_wait(blk_of(i_b), b)

    return k1(lut_t, tail_pairs)


def _sc_embed(x_flat, tbl, *, n_seq, n_batch, vocab):
    n_chunks = n_seq
    n_blocks = n_batch // CHUNK
    n_per_w = n_seq * CHUNK
    d_tiles = D_MODEL // TR

    mesh = plsc.VectorSubcoreMesh(core_axis_name="c", subcore_axis_name="s")

    @functools.partial(
        pl.kernel,
        mesh=mesh,
        out_type=jax.ShapeDtypeStruct(
            (n_seq, d_tiles, n_blocks, TR, CHUNK), jnp.float32
        ),
        scratch_types=[
            pltpu.VMEM((n_per_w,), jnp.int32),
            pltpu.VMEM((n_chunks, IPITCH), jnp.int32),
            pltpu.VMEM((NBUF, CHUNK, D_MODEL), jnp.float32),
            pltpu.VMEM((NBUF, d_tiles, TR, IPITCH), jnp.float32),
            pltpu.SemaphoreType.DMA((NBUF,)),
            pltpu.SemaphoreType.DMA((NBUF,)),
        ],
        compiler_params=pltpu.CompilerParams(
            use_tc_tiling_on_sc=False, needs_layout_passes=False
        ),
    )
    def k(tbl_hbm, idx_hbm, out_hbm, idx_v, idx2_v, raw_v, out_v, gsem, ssem):
        wid = lax.axis_index("s") * NC + lax.axis_index("c")
        pltpu.sync_copy(idx_hbm.at[pl.ds(wid * n_per_w, n_per_w)], idx_v)

        lane = lax.iota(jnp.int32, LANES)

        def repack(c):
            for q in range(CHUNK // LANES):
                pick = (lane + q * LANES) * n_seq + c
                idx2_v[c, pl.ds(q * LANES, LANES)] = plsc.load_gather(
                    idx_v, [pick]
                )

        def gather_start(c, b):
            pltpu.make_async_copy(
                tbl_hbm.at[idx2_v.at[c, pl.ds(0, CHUNK)]],
                raw_v.at[b], gsem.at[b],
            ).start()

        def gather_wait(c, b):
            pltpu.make_async_copy(
                tbl_hbm.at[idx2_v.at[c, pl.ds(0, CHUNK)]],
                raw_v.at[b], gsem.at[b],
            ).wait()

        def store_start(c, b):
            pltpu.make_async_copy(
                out_v.at[b, :, :, pl.ds(0, CHUNK)],
                out_hbm.at[c, :, wid], ssem.at[b],
            ).start()

        def store_wait(b):
            pltpu.make_async_copy(
                out_v.at[b, :, :, pl.ds(0, CHUNK)],
                out_hbm.at[0, :, 0], ssem.at[b],
            ).wait()

        for b in range(NBUF - 1):
            repack(b)
            gather_start(b, b)

        def group_body(g, _):
            for b in range(NBUF):
                c = g * NBUF + b

                @pl.when(c + NBUF - 1 < n_chunks)
                def _(c=c, b=b):
                    repack(c + NBUF - 1)
                    gather_start(c + NBUF - 1, (b + NBUF - 1) % NBUF)

                gather_wait(c, b)

                @pl.when(c >= NBUF)
                def _(b=b):
                    store_wait(b)

                @plsc.parallel_loop(0, CHUNK, step=1, unroll=4)
                def _(i, c=c, b=b):
                    col = jnp.full((LANES,), i, jnp.int32)
                    rot = plsc.load_gather(idx2_v.at[c], [col]) & RMASK
                    for grp in range(D_MODEL // LANES):
                        q = lane + (grp * LANES)
                        d = (q - rot) & DMASK
                        vals = raw_v[b, i, pl.ds(grp * LANES, LANES)]
                        plsc.store_scatter(
                            out_v.at[b],
                            [d // TR, d % TR, col],
                            vals,
                        )

                store_start(c, b)
            return ()

        lax.fori_loop(0, n_chunks // NBUF, group_body, ())

        for b in range(NBUF):
            store_wait(b)

    return k(tbl, x_flat)


def kernel(x, lut):
    n_batch, n_seq = x.shape
    vocab = lut.shape[0]
    x_flat = x.reshape(n_batch * n_seq).astype(jnp.int32)
    n_tail = vocab - (vocab // VBLK) * VBLK
    t = lut[vocab - n_tail:].reshape(n_tail // LANES, LANES, D_MODEL)
    tail_skewed = jnp.stack(
        [jnp.roll(t[:, r], r, axis=-1) for r in range(LANES)], axis=1
    ).reshape(n_tail, D_MODEL) * SCALE
    tail_pairs = tail_skewed.reshape(n_tail // 2, 2 * D_MODEL)
    tbl_pairs = _sc_detile(jnp.swapaxes(lut, 0, 1), tail_pairs, vocab=vocab)
    tbl = tbl_pairs.reshape(vocab * D_MODEL).reshape(vocab, D_MODEL)
    out5d = _sc_embed(
        x_flat, tbl, n_seq=n_seq, n_batch=n_batch, vocab=vocab
    )
    out = out5d.transpose(2, 4, 0, 1, 3).reshape(n_batch, n_seq, D_MODEL)
    return out
